# Initial kernel scaffold; baseline (speedup 1.0000x reference)
import jax
import jax.numpy as jnp
from jax.experimental import pallas as pl


def kernel(hidden_states, gate_kernel, gate_up_proj, down_proj):
    raise NotImplementedError("write your pallas kernel here")


# speedup vs baseline: 2.7710x; 2.7710x over previous
import jax
import jax.numpy as jnp
from jax.experimental import pallas as pl
from jax.experimental.pallas import tpu as pltpu

HIDDEN = 1024
INTER = 768
N_EXP = 64
N_TOK = 32


def _router_body(x_ref, gk_ref, logits_ref, wt_ref, aux_ref):
    x = x_ref[...]
    gk = gk_ref[...]
    logits = jax.lax.dot_general(
        x, gk, (((1,), (0,)), ((), ())), preferred_element_type=jnp.float32)
    logits_ref[...] = logits
    m = jnp.max(logits, axis=-1, keepdims=True)
    e = jnp.exp(logits - m)
    probs = e / jnp.sum(e, axis=-1, keepdims=True)

    iota = jax.lax.broadcasted_iota(jnp.int32, (N_TOK, N_EXP), 1)
    p1 = jnp.max(probs, axis=-1, keepdims=True)
    idx1 = jnp.min(jnp.where(probs == p1, iota, N_EXP), axis=-1, keepdims=True)
    masked = jnp.where(iota == idx1, -1.0, probs)
    p2 = jnp.max(masked, axis=-1, keepdims=True)
    idx2 = jnp.min(jnp.where(masked == p2, iota, N_EXP), axis=-1, keepdims=True)
    s = p1 + p2
    w1 = p1 / s
    w2 = p2 / s

    iota_e = jax.lax.broadcasted_iota(jnp.int32, (N_EXP, N_TOK), 0)
    sel1 = jnp.broadcast_to(idx1.reshape(1, N_TOK), (N_EXP, N_TOK))
    sel2 = jnp.broadcast_to(idx2.reshape(1, N_TOK), (N_EXP, N_TOK))
    w1b = jnp.broadcast_to(w1.reshape(1, N_TOK), (N_EXP, N_TOK))
    w2b = jnp.broadcast_to(w2.reshape(1, N_TOK), (N_EXP, N_TOK))
    wt = jnp.where(iota_e == sel1, w1b, 0.0) + jnp.where(iota_e == sel2, w2b, 0.0)
    wt_ref[...] = wt

    pm = jnp.mean(probs, axis=0)
    onehot = (jnp.where(iota == idx1, 1.0, 0.0)
              + jnp.where(iota == idx2, 1.0, 0.0))
    freq = jnp.mean(onehot, axis=0)
    aux = jnp.sum(pm * freq) * (N_EXP * 0.001)
    aux_ref[...] = jnp.broadcast_to(aux, (1, 1))


def _expert_body(x_ref, gup_ref, dwn_ref, wt_ref, out_ref):
    ei = pl.program_id(0)

    @pl.when(ei == 0)
    def _():
        out_ref[...] = jnp.zeros_like(out_ref)

    x = x_ref[...]
    gu = gup_ref[0]
    h = jax.lax.dot_general(
        x, gu, (((1,), (1,)), ((), ())), preferred_element_type=jnp.float32)
    g = h[:, :INTER]
    u = h[:, INTER:]
    inter = g * jax.nn.sigmoid(g) * u
    dw = dwn_ref[0]
    y = jax.lax.dot_general(
        inter, dw, (((1,), (1,)), ((), ())), preferred_element_type=jnp.float32)
    w = wt_ref[0]
    out_ref[...] += w * y


def kernel(hidden_states, gate_kernel, gate_up_proj, down_proj):
    batch, seq, hidden = hidden_states.shape
    x = hidden_states.reshape(N_TOK, HIDDEN)

    logits, wt, aux = pl.pallas_call(
        _router_body,
        out_shape=(
            jax.ShapeDtypeStruct((N_TOK, N_EXP), jnp.float32),
            jax.ShapeDtypeStruct((N_EXP, N_TOK), jnp.float32),
            jax.ShapeDtypeStruct((1, 1), jnp.float32),
        ),
    )(x, gate_kernel)

    wt3 = wt.reshape(N_EXP, N_TOK, 1)

    out = pl.pallas_call(
        _expert_body,
        grid=(N_EXP,),
        in_specs=[
            pl.BlockSpec((N_TOK, HIDDEN), lambda e: (0, 0)),
            pl.BlockSpec((1, 2 * INTER, HIDDEN), lambda e: (e, 0, 0)),
            pl.BlockSpec((1, HIDDEN, INTER), lambda e: (e, 0, 0)),
            pl.BlockSpec((1, N_TOK, 1), lambda e: (e, 0, 0)),
        ],
        out_specs=pl.BlockSpec((N_TOK, HIDDEN), lambda e: (0, 0)),
        out_shape=jax.ShapeDtypeStruct((N_TOK, HIDDEN), jnp.float32),
    )(x, gate_up_proj, down_proj, wt3)

    return (out.reshape(batch, seq, hidden), logits, aux[0, 0])


# speedup vs baseline: 3.3586x; 1.2121x over previous
import jax
import jax.numpy as jnp
from jax import lax
from jax.experimental import pallas as pl
from jax.experimental.pallas import tpu as pltpu
from jax.experimental.pallas import tpu_sc as plsc

HIDDEN = 1024
INTER = 768
N_EXP = 64
N_TOK = 32


def _router_body(x_ref, gk_ref, logits_ref, w_ref, wt_ref, aux_ref):
    x = x_ref[...]
    gk = gk_ref[...]
    logits = jax.lax.dot_general(
        x, gk, (((1,), (0,)), ((), ())), preferred_element_type=jnp.float32)
    logits_ref[...] = logits
    m = jnp.max(logits, axis=-1, keepdims=True)
    e = jnp.exp(logits - m)
    probs = e / jnp.sum(e, axis=-1, keepdims=True)

    iota = jax.lax.broadcasted_iota(jnp.int32, (N_TOK, N_EXP), 1)
    p1 = jnp.max(probs, axis=-1, keepdims=True)
    idx1 = jnp.min(jnp.where(probs == p1, iota, N_EXP), axis=-1, keepdims=True)
    masked = jnp.where(iota == idx1, -1.0, probs)
    p2 = jnp.max(masked, axis=-1, keepdims=True)
    idx2 = jnp.min(jnp.where(masked == p2, iota, N_EXP), axis=-1, keepdims=True)
    s = p1 + p2
    w1 = p1 / s
    w2 = p2 / s

    w_ref[...] = (jnp.where(iota == idx1, w1, 0.0)
                  + jnp.where(iota == idx2, w2, 0.0))

    iota_e = jax.lax.broadcasted_iota(jnp.int32, (N_EXP, N_TOK), 0)
    sel1 = jnp.broadcast_to(idx1.reshape(1, N_TOK), (N_EXP, N_TOK))
    sel2 = jnp.broadcast_to(idx2.reshape(1, N_TOK), (N_EXP, N_TOK))
    w1b = jnp.broadcast_to(w1.reshape(1, N_TOK), (N_EXP, N_TOK))
    w2b = jnp.broadcast_to(w2.reshape(1, N_TOK), (N_EXP, N_TOK))
    wt_ref[...] = (jnp.where(iota_e == sel1, w1b, 0.0)
                   + jnp.where(iota_e == sel2, w2b, 0.0))

    pm = jnp.mean(probs, axis=0)
    onehot = (jnp.where(iota == idx1, 1.0, 0.0)
              + jnp.where(iota == idx2, 1.0, 0.0))
    freq = jnp.mean(onehot, axis=0)
    aux = jnp.sum(pm * freq) * (N_EXP * 0.001)
    aux_ref[...] = jnp.broadcast_to(aux, (1, 1))


_LANES = 16


def _sc_dedupe_body(w_hbm, active_hbm, count_hbm, w_v, act_v, cnt_v):
    first = (lax.axis_index("c") == 0) & (lax.axis_index("s") == 0)

    @pl.when(first)
    def _():
        pltpu.sync_copy(w_hbm, w_v)

        acc = [jnp.zeros((_LANES,), jnp.float32) for _ in range(4)]
        for t in range(N_TOK):
            for c in range(4):
                acc[c] = jnp.maximum(acc[c], w_v[t, pl.ds(c * _LANES, _LANES)])

        for c in range(4):
            act_v[pl.ds(c * _LANES, _LANES)] = jnp.zeros((_LANES,), jnp.int32)

        zv = jnp.zeros((_LANES,), jnp.float32)
        ov = jnp.ones((_LANES,), jnp.int32)
        zi = jnp.zeros((_LANES,), jnp.int32)
        offset = jnp.int32(0)
        for c in range(4):
            pres = acc[c] > zv
            presi = jnp.where(pres, ov, zi)
            cs = plsc.cumsum(presi)
            pos = cs + offset - 1
            vals = lax.iota(jnp.int32, _LANES) + c * _LANES
            plsc.store_scatter(act_v, [pos], vals, mask=pres)
            offset = offset + jnp.sum(presi)

        cnt_v[...] = jnp.broadcast_to(offset, (_LANES,))
        pltpu.sync_copy(act_v, active_hbm)
        pltpu.sync_copy(cnt_v, count_hbm)


def _sc_dedupe(w):
    return pl.kernel(
        _sc_dedupe_body,
        out_type=(
            jax.ShapeDtypeStruct((N_EXP,), jnp.int32),
            jax.ShapeDtypeStruct((_LANES,), jnp.int32),
        ),
        mesh=plsc.VectorSubcoreMesh(
            core_axis_name="c", subcore_axis_name="s",
            num_cores=2, num_subcores=16),
        scratch_types=[
            pltpu.VMEM((N_TOK, N_EXP), jnp.float32),
            pltpu.VMEM((N_EXP,), jnp.int32),
            pltpu.VMEM((_LANES,), jnp.int32),
        ],
        compiler_params=pltpu.CompilerParams(needs_layout_passes=False),
    )(w)


def _expert_body(active_ref, count_ref, x_ref, gup_ref, dwn_ref, wt_ref,
                 out_ref):
    i = pl.program_id(0)

    @pl.when(i == 0)
    def _():
        out_ref[...] = jnp.zeros_like(out_ref)

    @pl.when(i < count_ref[0])
    def _():
        x = x_ref[...]
        gu = gup_ref[0]
        h = jax.lax.dot_general(
            x, gu, (((1,), (1,)), ((), ())),
            preferred_element_type=jnp.float32)
        g = h[:, :INTER]
        u = h[:, INTER:]
        inter = g * jax.nn.sigmoid(g) * u
        dw = dwn_ref[0]
        y = jax.lax.dot_general(
            inter, dw, (((1,), (1,)), ((), ())),
            preferred_element_type=jnp.float32)
        w = wt_ref[0]
        out_ref[...] += w * y


def kernel(hidden_states, gate_kernel, gate_up_proj, down_proj):
    batch, seq, hidden = hidden_states.shape
    x = hidden_states.reshape(N_TOK, HIDDEN)

    logits, w_tok, wt, aux = pl.pallas_call(
        _router_body,
        out_shape=(
            jax.ShapeDtypeStruct((N_TOK, N_EXP), jnp.float32),
            jax.ShapeDtypeStruct((N_TOK, N_EXP), jnp.float32),
            jax.ShapeDtypeStruct((N_EXP, N_TOK), jnp.float32),
            jax.ShapeDtypeStruct((1, 1), jnp.float32),
        ),
    )(x, gate_kernel)

    active, count = _sc_dedupe(w_tok)

    wt3 = wt.reshape(N_EXP, N_TOK, 1)

    def _widx(i, a_ref, c_ref):
        return (a_ref[jnp.minimum(i, c_ref[0] - 1)], 0, 0)

    grid_spec = pltpu.PrefetchScalarGridSpec(
        num_scalar_prefetch=2,
        grid=(N_EXP,),
        in_specs=[
            pl.BlockSpec((N_TOK, HIDDEN), lambda i, a, c: (0, 0)),
            pl.BlockSpec((1, 2 * INTER, HIDDEN), _widx),
            pl.BlockSpec((1, HIDDEN, INTER), _widx),
            pl.BlockSpec((1, N_TOK, 1), _widx),
        ],
        out_specs=pl.BlockSpec((N_TOK, HIDDEN), lambda i, a, c: (0, 0)),
    )

    out = pl.pallas_call(
        _expert_body,
        grid_spec=grid_spec,
        out_shape=jax.ShapeDtypeStruct((N_TOK, HIDDEN), jnp.float32),
    )(active, count, x, gate_up_proj, down_proj, wt3)

    return (out.reshape(batch, seq, hidden), logits, aux[0, 0])


# speedup vs baseline: 3.4595x; 1.0301x over previous
import jax
import jax.numpy as jnp
from jax import lax
from jax.experimental import pallas as pl
from jax.experimental.pallas import tpu as pltpu
from jax.experimental.pallas import tpu_sc as plsc

HIDDEN = 1024
INTER = 768
N_EXP = 64
N_TOK = 32


def _router_body(x_ref, gk_ref, logits_ref, w_ref, wt_ref, aux_ref):
    x = x_ref[...]
    gk = gk_ref[...]
    logits = jax.lax.dot_general(
        x, gk, (((1,), (0,)), ((), ())), preferred_element_type=jnp.float32)
    logits_ref[...] = logits
    m = jnp.max(logits, axis=-1, keepdims=True)
    e = jnp.exp(logits - m)
    probs = e / jnp.sum(e, axis=-1, keepdims=True)

    iota = jax.lax.broadcasted_iota(jnp.int32, (N_TOK, N_EXP), 1)
    p1 = jnp.max(probs, axis=-1, keepdims=True)
    idx1 = jnp.min(jnp.where(probs == p1, iota, N_EXP), axis=-1, keepdims=True)
    masked = jnp.where(iota == idx1, -1.0, probs)
    p2 = jnp.max(masked, axis=-1, keepdims=True)
    idx2 = jnp.min(jnp.where(masked == p2, iota, N_EXP), axis=-1, keepdims=True)
    s = p1 + p2
    w1 = p1 / s
    w2 = p2 / s

    w_ref[...] = (jnp.where(iota == idx1, w1, 0.0)
                  + jnp.where(iota == idx2, w2, 0.0))

    iota_e = jax.lax.broadcasted_iota(jnp.int32, (N_EXP, N_TOK), 0)
    sel1 = jnp.broadcast_to(idx1.reshape(1, N_TOK), (N_EXP, N_TOK))
    sel2 = jnp.broadcast_to(idx2.reshape(1, N_TOK), (N_EXP, N_TOK))
    w1b = jnp.broadcast_to(w1.reshape(1, N_TOK), (N_EXP, N_TOK))
    w2b = jnp.broadcast_to(w2.reshape(1, N_TOK), (N_EXP, N_TOK))
    wt_ref[...] = (jnp.where(iota_e == sel1, w1b, 0.0)
                   + jnp.where(iota_e == sel2, w2b, 0.0))

    pm = jnp.mean(probs, axis=0)
    onehot = (jnp.where(iota == idx1, 1.0, 0.0)
              + jnp.where(iota == idx2, 1.0, 0.0))
    freq = jnp.mean(onehot, axis=0)
    aux = jnp.sum(pm * freq) * (N_EXP * 0.001)
    aux_ref[...] = jnp.broadcast_to(aux, (1, 1))


_LANES = 16


def _sc_dedupe_body(w_hbm, active_hbm, count_hbm, w_v, act_v, cnt_v):
    first = (lax.axis_index("c") == 0) & (lax.axis_index("s") == 0)

    @pl.when(first)
    def _():
        pltpu.sync_copy(w_hbm, w_v)

        acc = [jnp.zeros((_LANES,), jnp.float32) for _ in range(4)]
        for t in range(N_TOK):
            for c in range(4):
                acc[c] = jnp.maximum(acc[c], w_v[t, pl.ds(c * _LANES, _LANES)])

        for c in range(4):
            act_v[pl.ds(c * _LANES, _LANES)] = jnp.zeros((_LANES,), jnp.int32)

        zv = jnp.zeros((_LANES,), jnp.float32)
        ov = jnp.ones((_LANES,), jnp.int32)
        zi = jnp.zeros((_LANES,), jnp.int32)
        offset = jnp.int32(0)
        for c in range(4):
            pres = acc[c] > zv
            presi = jnp.where(pres, ov, zi)
            cs = plsc.cumsum(presi)
            pos = cs + offset - 1
            vals = lax.iota(jnp.int32, _LANES) + c * _LANES
            plsc.store_scatter(act_v, [pos], vals, mask=pres)
            offset = offset + jnp.sum(presi)

        cnt_v[...] = jnp.broadcast_to(offset, (_LANES,))
        pltpu.sync_copy(act_v, active_hbm)
        pltpu.sync_copy(cnt_v, count_hbm)


def _sc_dedupe(w):
    return pl.kernel(
        _sc_dedupe_body,
        out_type=(
            jax.ShapeDtypeStruct((N_EXP,), jnp.int32),
            jax.ShapeDtypeStruct((_LANES,), jnp.int32),
        ),
        mesh=plsc.VectorSubcoreMesh(
            core_axis_name="c", subcore_axis_name="s",
            num_cores=2, num_subcores=16),
        scratch_types=[
            pltpu.VMEM((N_TOK, N_EXP), jnp.float32),
            pltpu.VMEM((N_EXP,), jnp.int32),
            pltpu.VMEM((_LANES,), jnp.int32),
        ],
        compiler_params=pltpu.CompilerParams(needs_layout_passes=False),
    )(w)


def _expert_body(active_ref, count_ref, x_ref, gup_ref, dwn_ref, wt_ref,
                 out_ref):
    i = pl.program_id(0)

    @pl.when(i == 0)
    def _():
        out_ref[...] = jnp.zeros_like(out_ref)

    x = x_ref[...]
    gu = gup_ref[0]
    h = jax.lax.dot_general(
        x, gu, (((1,), (1,)), ((), ())),
        preferred_element_type=jnp.float32)
    g = h[:, :INTER]
    u = h[:, INTER:]
    inter = g * jax.nn.sigmoid(g) * u
    dw = dwn_ref[0]
    y = jax.lax.dot_general(
        inter, dw, (((1,), (1,)), ((), ())),
        preferred_element_type=jnp.float32)
    w = wt_ref[0]
    out_ref[...] += w * y


def kernel(hidden_states, gate_kernel, gate_up_proj, down_proj):
    batch, seq, hidden = hidden_states.shape
    x = hidden_states.reshape(N_TOK, HIDDEN)

    logits, w_tok, wt, aux = pl.pallas_call(
        _router_body,
        out_shape=(
            jax.ShapeDtypeStruct((N_TOK, N_EXP), jnp.float32),
            jax.ShapeDtypeStruct((N_TOK, N_EXP), jnp.float32),
            jax.ShapeDtypeStruct((N_EXP, N_TOK), jnp.float32),
            jax.ShapeDtypeStruct((1, 1), jnp.float32),
        ),
    )(x, gate_kernel)

    active, count = _sc_dedupe(w_tok)

    wt3 = wt.reshape(N_EXP, N_TOK, 1)

    def _widx(i, a_ref, c_ref):
        return (a_ref[i], 0, 0)

    grid_spec = pltpu.PrefetchScalarGridSpec(
        num_scalar_prefetch=2,
        grid=(count[0],),
        in_specs=[
            pl.BlockSpec((N_TOK, HIDDEN), lambda i, a, c: (0, 0)),
            pl.BlockSpec((1, 2 * INTER, HIDDEN), _widx),
            pl.BlockSpec((1, HIDDEN, INTER), _widx),
            pl.BlockSpec((1, N_TOK, 1), _widx),
        ],
        out_specs=pl.BlockSpec((N_TOK, HIDDEN), lambda i, a, c: (0, 0)),
    )

    out = pl.pallas_call(
        _expert_body,
        grid_spec=grid_spec,
        out_shape=jax.ShapeDtypeStruct((N_TOK, HIDDEN), jnp.float32),
    )(active, count, x, gate_up_proj, down_proj, wt3)

    return (out.reshape(batch, seq, hidden), logits, aux[0, 0])


# speedup vs baseline: 3.8378x; 1.1093x over previous
import jax
import jax.numpy as jnp
from jax import lax
from jax.experimental import pallas as pl
from jax.experimental.pallas import tpu as pltpu
from jax.experimental.pallas import tpu_sc as plsc

HIDDEN = 1024
INTER = 768
N_EXP = 64
N_TOK = 32


def _router_body(x_ref, gk_ref, logits_ref, w_ref, wt_ref, aux_ref,
                 act_ref, cnt_ref):
    x = x_ref[...]
    gk = gk_ref[...]
    logits = jax.lax.dot_general(
        x, gk, (((1,), (0,)), ((), ())), preferred_element_type=jnp.float32)
    logits_ref[...] = logits
    m = jnp.max(logits, axis=-1, keepdims=True)
    e = jnp.exp(logits - m)
    probs = e / jnp.sum(e, axis=-1, keepdims=True)

    iota = jax.lax.broadcasted_iota(jnp.int32, (N_TOK, N_EXP), 1)
    p1 = jnp.max(probs, axis=-1, keepdims=True)
    idx1 = jnp.min(jnp.where(probs == p1, iota, N_EXP), axis=-1, keepdims=True)
    masked = jnp.where(iota == idx1, -1.0, probs)
    p2 = jnp.max(masked, axis=-1, keepdims=True)
    idx2 = jnp.min(jnp.where(masked == p2, iota, N_EXP), axis=-1, keepdims=True)
    s = p1 + p2
    w1 = p1 / s
    w2 = p2 / s

    w_ref[...] = (jnp.where(iota == idx1, w1, 0.0)
                  + jnp.where(iota == idx2, w2, 0.0))

    iota_e = jax.lax.broadcasted_iota(jnp.int32, (N_EXP, N_TOK), 0)
    sel1 = jnp.broadcast_to(idx1.reshape(1, N_TOK), (N_EXP, N_TOK))
    sel2 = jnp.broadcast_to(idx2.reshape(1, N_TOK), (N_EXP, N_TOK))
    w1b = jnp.broadcast_to(w1.reshape(1, N_TOK), (N_EXP, N_TOK))
    w2b = jnp.broadcast_to(w2.reshape(1, N_TOK), (N_EXP, N_TOK))
    wt_ref[...] = (jnp.where(iota_e == sel1, w1b, 0.0)
                   + jnp.where(iota_e == sel2, w2b, 0.0))

    pm = jnp.mean(probs, axis=0)
    onehot = (jnp.where(iota == idx1, 1.0, 0.0)
              + jnp.where(iota == idx2, 1.0, 0.0))
    freq = jnp.mean(onehot, axis=0)
    aux = jnp.sum(pm * freq) * (N_EXP * 0.001)
    aux_ref[...] = jnp.broadcast_to(aux, (1, 1))

    pres = jnp.max(onehot, axis=0, keepdims=True)
    iota_r = jax.lax.broadcasted_iota(jnp.int32, (N_EXP, N_EXP), 0)
    iota_c = jax.lax.broadcasted_iota(jnp.int32, (N_EXP, N_EXP), 1)
    tri = jnp.where(iota_r <= iota_c, 1.0, 0.0)
    csum = jax.lax.dot_general(
        pres, tri, (((1,), (0,)), ((), ())),
        preferred_element_type=jnp.float32)
    csum_b = jnp.broadcast_to(csum, (N_EXP, N_EXP))
    pres_b = jnp.broadcast_to(pres, (N_EXP, N_EXP))
    hit = jnp.where((csum_b == (iota_r + 1).astype(jnp.float32))
                    & (pres_b > 0.0), 1.0, 0.0)
    act = jnp.sum(hit * iota_c.astype(jnp.float32), axis=1)
    act_ref[...] = act.astype(jnp.int32)
    cnt_ref[...] = jnp.broadcast_to(jnp.sum(pres), (16,)).astype(jnp.int32)


_LANES = 16


def _sc_dedupe_body(w_hbm, active_hbm, count_hbm, w_v, act_v, cnt_v):
    first = (lax.axis_index("c") == 0) & (lax.axis_index("s") == 0)

    @pl.when(first)
    def _():
        pltpu.sync_copy(w_hbm, w_v)

        acc = [jnp.zeros((_LANES,), jnp.float32) for _ in range(4)]
        for t in range(N_TOK):
            for c in range(4):
                acc[c] = jnp.maximum(acc[c], w_v[t, pl.ds(c * _LANES, _LANES)])

        for c in range(4):
            act_v[pl.ds(c * _LANES, _LANES)] = jnp.zeros((_LANES,), jnp.int32)

        zv = jnp.zeros((_LANES,), jnp.float32)
        ov = jnp.ones((_LANES,), jnp.int32)
        zi = jnp.zeros((_LANES,), jnp.int32)
        offset = jnp.int32(0)
        for c in range(4):
            pres = acc[c] > zv
            presi = jnp.where(pres, ov, zi)
            cs = plsc.cumsum(presi)
            pos = cs + offset - 1
            vals = lax.iota(jnp.int32, _LANES) + c * _LANES
            plsc.store_scatter(act_v, [pos], vals, mask=pres)
            offset = offset + jnp.sum(presi)

        cnt_v[...] = jnp.broadcast_to(offset, (_LANES,))
        pltpu.sync_copy(act_v, active_hbm)
        pltpu.sync_copy(cnt_v, count_hbm)


def _sc_dedupe(w):
    return pl.kernel(
        _sc_dedupe_body,
        out_type=(
            jax.ShapeDtypeStruct((N_EXP,), jnp.int32),
            jax.ShapeDtypeStruct((_LANES,), jnp.int32),
        ),
        mesh=plsc.VectorSubcoreMesh(
            core_axis_name="c", subcore_axis_name="s",
            num_cores=2, num_subcores=16),
        scratch_types=[
            pltpu.VMEM((N_TOK, N_EXP), jnp.float32),
            pltpu.VMEM((N_EXP,), jnp.int32),
            pltpu.VMEM((_LANES,), jnp.int32),
        ],
        compiler_params=pltpu.CompilerParams(needs_layout_passes=False),
    )(w)


def _expert_body(active_ref, count_ref, x_ref, gup_ref, dwn_ref, wt_ref,
                 out_ref):
    i = pl.program_id(0)

    @pl.when(i == 0)
    def _():
        out_ref[...] = jnp.zeros_like(out_ref)

    x = x_ref[...]
    gu = gup_ref[0]
    h = jax.lax.dot_general(
        x, gu, (((1,), (1,)), ((), ())),
        preferred_element_type=jnp.float32)
    g = h[:, :INTER]
    u = h[:, INTER:]
    inter = g * jax.nn.sigmoid(g) * u
    dw = dwn_ref[0]
    y = jax.lax.dot_general(
        inter, dw, (((1,), (1,)), ((), ())),
        preferred_element_type=jnp.float32)
    w = wt_ref[0]
    out_ref[...] += w * y


def kernel(hidden_states, gate_kernel, gate_up_proj, down_proj):
    batch, seq, hidden = hidden_states.shape
    x = hidden_states.reshape(N_TOK, HIDDEN)

    logits, w_tok, wt, aux, active, count = pl.pallas_call(
        _router_body,
        out_shape=(
            jax.ShapeDtypeStruct((N_TOK, N_EXP), jnp.float32),
            jax.ShapeDtypeStruct((N_TOK, N_EXP), jnp.float32),
            jax.ShapeDtypeStruct((N_EXP, N_TOK), jnp.float32),
            jax.ShapeDtypeStruct((1, 1), jnp.float32),
            jax.ShapeDtypeStruct((N_EXP,), jnp.int32),
            jax.ShapeDtypeStruct((_LANES,), jnp.int32),
        ),
    )(x, gate_kernel)

    wt3 = wt.reshape(N_EXP, N_TOK, 1)

    def _widx(i, a_ref, c_ref):
        return (a_ref[i], 0, 0)

    grid_spec = pltpu.PrefetchScalarGridSpec(
        num_scalar_prefetch=2,
        grid=(count[0],),
        in_specs=[
            pl.BlockSpec((N_TOK, HIDDEN), lambda i, a, c: (0, 0)),
            pl.BlockSpec((1, 2 * INTER, HIDDEN), _widx),
            pl.BlockSpec((1, HIDDEN, INTER), _widx),
            pl.BlockSpec((1, N_TOK, 1), _widx),
        ],
        out_specs=pl.BlockSpec((N_TOK, HIDDEN), lambda i, a, c: (0, 0)),
    )

    out = pl.pallas_call(
        _expert_body,
        grid_spec=grid_spec,
        out_shape=jax.ShapeDtypeStruct((N_TOK, HIDDEN), jnp.float32),
    )(active, count, x, gate_up_proj, down_proj, wt3)

    return (out.reshape(batch, seq, hidden), logits, aux[0, 0])
